# initial kernel scaffold (unmeasured)
import jax
import jax.numpy as jnp
from jax import lax
from jax.experimental import pallas as pl
from jax.experimental.pallas import tpu as pltpu

N_DEV = 4
M_PER = 1024
N_COLS = 8192
HALF = N_COLS // 2
TILE = 2048
NT = HALF // TILE

_GC = 0.7978845608028654


def _gelu(y):
    return 0.5 * y * (1.0 + jnp.tanh(_GC * (y + 0.044715 * y * y * y)))


def kernel(x, w_mat):
    x = x.astype(jnp.bfloat16)
    w_mat = w_mat.astype(jnp.bfloat16)
    m, _ = x.shape

    def body(x_ref, w_ref, out_ref, cw_ref, ccw_ref,
             cw_send, cw_recv, ccw_send, ccw_recv, out_sem):
        p = lax.axis_index("i")
        right = lax.rem(p + 1, N_DEV)
        left = lax.rem(p + N_DEV - 1, N_DEV)

        barrier = pltpu.get_barrier_semaphore()
        for nbr in (left, right):
            pl.semaphore_signal(barrier, inc=1, device_id=(nbr,),
                                device_id_type=pl.DeviceIdType.MESH)
        pl.semaphore_wait(barrier, 2)

        def partial_tile(chunk, half, j):
            xc = x_ref[pl.ds(chunk * M_PER, M_PER), :]
            w_t = w_ref[:, half * HALF + j * TILE: half * HALF + (j + 1) * TILE]
            return jnp.dot(xc, w_t, preferred_element_type=jnp.float32)

        c_cw0 = lax.rem(p + N_DEV - 1, N_DEV)
        c_ccw0 = lax.rem(p + 1, N_DEV)
        for j in range(NT):
            sl = slice(j * TILE, (j + 1) * TILE)
            cw_ref[0, :, sl] = partial_tile(c_cw0, 0, j).astype(jnp.bfloat16)
            ccw_ref[0, :, sl] = partial_tile(c_ccw0, 1, j).astype(jnp.bfloat16)

        for h in range(N_DEV - 1):
            s_slot, r_slot = h % 2, (h + 1) % 2
            cw_rdma = pltpu.make_async_remote_copy(
                src_ref=cw_ref.at[s_slot], dst_ref=cw_ref.at[r_slot],
                send_sem=cw_send.at[h], recv_sem=cw_recv.at[h],
                device_id=(right,), device_id_type=pl.DeviceIdType.MESH)
            ccw_rdma = pltpu.make_async_remote_copy(
                src_ref=ccw_ref.at[s_slot], dst_ref=ccw_ref.at[r_slot],
                send_sem=ccw_send.at[h], recv_sem=ccw_recv.at[h],
                device_id=(left,), device_id_type=pl.DeviceIdType.MESH)
            cw_rdma.start()
            ccw_rdma.start()
            cw_rdma.wait()
            ccw_rdma.wait()

            c_cw = lax.rem(p + 2 * N_DEV - 2 - h, N_DEV)
            c_ccw = lax.rem(p + 2 + h, N_DEV)
            if h < N_DEV - 2:
                for j in range(NT):
                    sl = slice(j * TILE, (j + 1) * TILE)
                    cw_ref[r_slot, :, sl] = (
                        cw_ref[r_slot, :, sl].astype(jnp.float32)
                        + partial_tile(c_cw, 0, j)
                    ).astype(jnp.bfloat16)
                    ccw_ref[r_slot, :, sl] = (
                        ccw_ref[r_slot, :, sl].astype(jnp.float32)
                        + partial_tile(c_ccw, 1, j)
                    ).astype(jnp.bfloat16)
            else:
                for j in range(NT):
                    sl = slice(j * TILE, (j + 1) * TILE)
                    y_l = (cw_ref[r_slot, :, sl].astype(jnp.float32)
                           + partial_tile(p, 0, j))
                    cw_ref[s_slot, :, sl] = _gelu(y_l).astype(jnp.bfloat16)
                    y_r = (ccw_ref[r_slot, :, sl].astype(jnp.float32)
                           + partial_tile(p, 1, j))
                    ccw_ref[s_slot, :, sl] = _gelu(y_r).astype(jnp.bfloat16)
                cp_l = pltpu.make_async_copy(
                    cw_ref.at[s_slot], out_ref.at[:, 0:HALF], out_sem.at[0])
                cp_r = pltpu.make_async_copy(
                    ccw_ref.at[s_slot], out_ref.at[:, HALF:N_COLS],
                    out_sem.at[1])
                cp_l.start()
                cp_r.start()
                cp_l.wait()
                cp_r.wait()

    out_shape = jax.ShapeDtypeStruct((M_PER, N_COLS), jnp.bfloat16)
    return pl.pallas_call(
        body,
        out_shape=out_shape,
        in_specs=[pl.BlockSpec(memory_space=pltpu.VMEM),
                  pl.BlockSpec(memory_space=pltpu.VMEM)],
        out_specs=pl.BlockSpec(memory_space=pltpu.ANY),
        scratch_shapes=[
            pltpu.VMEM((2, M_PER, HALF), jnp.bfloat16),
            pltpu.VMEM((2, M_PER, HALF), jnp.bfloat16),
            pltpu.SemaphoreType.DMA((N_DEV - 1,)),
            pltpu.SemaphoreType.DMA((N_DEV - 1,)),
            pltpu.SemaphoreType.DMA((N_DEV - 1,)),
            pltpu.SemaphoreType.DMA((N_DEV - 1,)),
            pltpu.SemaphoreType.DMA((2,)),
        ],
        compiler_params=pltpu.CompilerParams(collective_id=0),
    )(x, w_mat)


# baseline (device time: 416057 ns/iter reference)
import jax
import jax.numpy as jnp
from jax import lax
from jax.experimental import pallas as pl
from jax.experimental.pallas import tpu as pltpu

N_DEV = 4
M_PER = 1024
N_COLS = 8192
HALF = N_COLS // 2
TILE = 1024
NT = HALF // TILE

_GC = 0.7978845608028654


def _gelu(y):
    return 0.5 * y * (1.0 + jnp.tanh(_GC * (y + 0.044715 * y * y * y)))


def kernel(x, w_mat):
    x = x.astype(jnp.bfloat16)
    w_mat = w_mat.astype(jnp.bfloat16)
    m, _ = x.shape

    def body(x_ref, w_ref, out_ref, cw_ref, ccw_ref,
             cw_send, cw_recv, ccw_send, ccw_recv, out_sem):
        p = lax.axis_index("i")
        right = lax.rem(p + 1, N_DEV)
        left = lax.rem(p + N_DEV - 1, N_DEV)

        barrier = pltpu.get_barrier_semaphore()
        for nbr in (left, right):
            pl.semaphore_signal(barrier, inc=1, device_id=(nbr,),
                                device_id_type=pl.DeviceIdType.MESH)
        pl.semaphore_wait(barrier, 2)

        def partial_tile(chunk, half, j):
            xc = x_ref[pl.ds(chunk * M_PER, M_PER), :]
            w_t = w_ref[:, half * HALF + j * TILE: half * HALF + (j + 1) * TILE]
            return jnp.dot(xc, w_t, preferred_element_type=jnp.float32)

        c_cw0 = lax.rem(p + N_DEV - 1, N_DEV)
        c_ccw0 = lax.rem(p + 1, N_DEV)
        for j in range(NT):
            sl = slice(j * TILE, (j + 1) * TILE)
            cw_ref[0, :, sl] = partial_tile(c_cw0, 0, j).astype(jnp.bfloat16)
            ccw_ref[0, :, sl] = partial_tile(c_ccw0, 1, j).astype(jnp.bfloat16)

        for h in range(N_DEV - 1):
            s_slot, r_slot = h % 2, (h + 1) % 2
            cw_rdma = pltpu.make_async_remote_copy(
                src_ref=cw_ref.at[s_slot], dst_ref=cw_ref.at[r_slot],
                send_sem=cw_send.at[h], recv_sem=cw_recv.at[h],
                device_id=(right,), device_id_type=pl.DeviceIdType.MESH)
            ccw_rdma = pltpu.make_async_remote_copy(
                src_ref=ccw_ref.at[s_slot], dst_ref=ccw_ref.at[r_slot],
                send_sem=ccw_send.at[h], recv_sem=ccw_recv.at[h],
                device_id=(left,), device_id_type=pl.DeviceIdType.MESH)
            cw_rdma.start()
            ccw_rdma.start()
            cw_rdma.wait()
            ccw_rdma.wait()

            c_cw = lax.rem(p + 2 * N_DEV - 2 - h, N_DEV)
            c_ccw = lax.rem(p + 2 + h, N_DEV)
            if h < N_DEV - 2:
                for j in range(NT):
                    sl = slice(j * TILE, (j + 1) * TILE)
                    cw_ref[r_slot, :, sl] = (
                        cw_ref[r_slot, :, sl].astype(jnp.float32)
                        + partial_tile(c_cw, 0, j)
                    ).astype(jnp.bfloat16)
                    ccw_ref[r_slot, :, sl] = (
                        ccw_ref[r_slot, :, sl].astype(jnp.float32)
                        + partial_tile(c_ccw, 1, j)
                    ).astype(jnp.bfloat16)
            else:
                for j in range(NT):
                    sl = slice(j * TILE, (j + 1) * TILE)
                    y_l = (cw_ref[r_slot, :, sl].astype(jnp.float32)
                           + partial_tile(p, 0, j))
                    cw_ref[s_slot, :, sl] = _gelu(y_l).astype(jnp.bfloat16)
                    y_r = (ccw_ref[r_slot, :, sl].astype(jnp.float32)
                           + partial_tile(p, 1, j))
                    ccw_ref[s_slot, :, sl] = _gelu(y_r).astype(jnp.bfloat16)
                cp_l = pltpu.make_async_copy(
                    cw_ref.at[s_slot], out_ref.at[:, 0:HALF], out_sem.at[0])
                cp_r = pltpu.make_async_copy(
                    ccw_ref.at[s_slot], out_ref.at[:, HALF:N_COLS],
                    out_sem.at[1])
                cp_l.start()
                cp_r.start()
                cp_l.wait()
                cp_r.wait()

    out_shape = jax.ShapeDtypeStruct((M_PER, N_COLS), jnp.bfloat16)
    return pl.pallas_call(
        body,
        out_shape=out_shape,
        in_specs=[pl.BlockSpec(memory_space=pltpu.VMEM),
                  pl.BlockSpec(memory_space=pltpu.VMEM)],
        out_specs=pl.BlockSpec(memory_space=pl.ANY),
        scratch_shapes=[
            pltpu.VMEM((2, M_PER, HALF), jnp.bfloat16),
            pltpu.VMEM((2, M_PER, HALF), jnp.bfloat16),
            pltpu.SemaphoreType.DMA((N_DEV - 1,)),
            pltpu.SemaphoreType.DMA((N_DEV - 1,)),
            pltpu.SemaphoreType.DMA((N_DEV - 1,)),
            pltpu.SemaphoreType.DMA((N_DEV - 1,)),
            pltpu.SemaphoreType.DMA((2,)),
        ],
        compiler_params=pltpu.CompilerParams(
            collective_id=0, vmem_limit_bytes=64 * 1024 * 1024),
    )(x, w_mat)


# device time: 335018 ns/iter; 1.2419x vs baseline; 1.2419x over previous
import jax
import jax.numpy as jnp
from jax import lax
from jax.experimental import pallas as pl
from jax.experimental.pallas import tpu as pltpu

N_DEV = 4
M_PER = 1024
N_COLS = 8192
HALF = N_COLS // 2
TILE = 1024
NT = HALF // TILE
N_HOP = N_DEV - 1

_GC = 0.7978845608028654


def _gelu(y):
    return 0.5 * y * (1.0 + jnp.tanh(_GC * (y + 0.044715 * y * y * y)))


def _ts(j):
    return slice(j * TILE, (j + 1) * TILE)


def kernel(x, w_mat):
    x = x.astype(jnp.bfloat16)
    w_mat = w_mat.astype(jnp.bfloat16)

    def body(x_ref, w_ref, out_ref, cw_ref, ccw_ref,
             cw_send, cw_recv, ccw_send, ccw_recv,
             cw_credit, ccw_credit, out_sem):
        p = lax.axis_index("i")
        right = lax.rem(p + 1, N_DEV)
        left = lax.rem(p + N_DEV - 1, N_DEV)

        barrier = pltpu.get_barrier_semaphore()
        for nbr in (left, right):
            pl.semaphore_signal(barrier, inc=1, device_id=(nbr,),
                                device_id_type=pl.DeviceIdType.MESH)
        pl.semaphore_wait(barrier, 2)

        def partial_tile(chunk, half, j):
            xc = x_ref[pl.ds(chunk * M_PER, M_PER), :]
            c0 = half * HALF + j * TILE
            return jnp.dot(xc, w_ref[:, c0:c0 + TILE],
                           preferred_element_type=jnp.float32)

        def rdma(ring, h, j):
            ref, ssem, rsem, dst = (
                (cw_ref, cw_send, cw_recv, right) if ring == 0
                else (ccw_ref, ccw_send, ccw_recv, left))
            return pltpu.make_async_remote_copy(
                src_ref=ref.at[h % 2, :, _ts(j)],
                dst_ref=ref.at[(h + 1) % 2, :, _ts(j)],
                send_sem=ssem.at[h, j], recv_sem=rsem.at[h, j],
                device_id=(dst,), device_id_type=pl.DeviceIdType.MESH)

        def give_credit(ring, h, j):
            sem, upstream = ((cw_credit, left) if ring == 0
                             else (ccw_credit, right))
            pl.semaphore_signal(sem.at[h, j], inc=1, device_id=(upstream,),
                                device_id_type=pl.DeviceIdType.MESH)

        def take_credit(ring, h, j):
            sem = cw_credit if ring == 0 else ccw_credit
            pl.semaphore_wait(sem.at[h, j], 1)

        rings = ((0, cw_ref), (1, ccw_ref))

        c0 = {0: lax.rem(p + N_DEV - 1, N_DEV), 1: lax.rem(p + 1, N_DEV)}
        for j in range(NT):
            for ring, ref in rings:
                ref[0, :, _ts(j)] = (
                    partial_tile(c0[ring], ring, j).astype(jnp.bfloat16))
                rdma(ring, 0, j).start()

        c1 = lax.rem(p + 2, N_DEV)
        for j in range(NT):
            for ring, ref in rings:
                d = rdma(ring, 0, j)
                d.wait_recv()
                ref[1, :, _ts(j)] = (
                    ref[1, :, _ts(j)].astype(jnp.float32)
                    + partial_tile(c1, ring, j)).astype(jnp.bfloat16)
                d.wait_send()
                give_credit(ring, 1, j)
                take_credit(ring, 1, j)
                rdma(ring, 1, j).start()

        c2 = {0: lax.rem(p + 1, N_DEV), 1: lax.rem(p + 3, N_DEV)}
        for j in range(NT):
            for ring, ref in rings:
                d = rdma(ring, 1, j)
                d.wait_recv()
                d.wait_send()
                give_credit(ring, 2, j)
                ref[0, :, _ts(j)] = (
                    ref[0, :, _ts(j)].astype(jnp.float32)
                    + partial_tile(c2[ring], ring, j)).astype(jnp.bfloat16)
                take_credit(ring, 2, j)
                rdma(ring, 2, j).start()

        for j in range(NT):
            for ring, ref in rings:
                d = rdma(ring, 2, j)
                d.wait_recv()
                d.wait_send()
                y = (ref[1, :, _ts(j)].astype(jnp.float32)
                     + partial_tile(p, ring, j))
                ref[0, :, _ts(j)] = _gelu(y).astype(jnp.bfloat16)
                col = ring * HALF + j * TILE
                pltpu.make_async_copy(
                    ref.at[0, :, _ts(j)],
                    out_ref.at[:, col:col + TILE],
                    out_sem.at[ring, j]).start()

        for j in range(NT):
            for ring, ref in rings:
                col = ring * HALF + j * TILE
                pltpu.make_async_copy(
                    ref.at[0, :, _ts(j)],
                    out_ref.at[:, col:col + TILE],
                    out_sem.at[ring, j]).wait()

    out_shape = jax.ShapeDtypeStruct((M_PER, N_COLS), jnp.bfloat16)
    return pl.pallas_call(
        body,
        out_shape=out_shape,
        in_specs=[pl.BlockSpec(memory_space=pltpu.VMEM),
                  pl.BlockSpec(memory_space=pltpu.VMEM)],
        out_specs=pl.BlockSpec(memory_space=pl.ANY),
        scratch_shapes=[
            pltpu.VMEM((2, M_PER, HALF), jnp.bfloat16),
            pltpu.VMEM((2, M_PER, HALF), jnp.bfloat16),
            pltpu.SemaphoreType.DMA((N_HOP, NT)),
            pltpu.SemaphoreType.DMA((N_HOP, NT)),
            pltpu.SemaphoreType.DMA((N_HOP, NT)),
            pltpu.SemaphoreType.DMA((N_HOP, NT)),
            pltpu.SemaphoreType.REGULAR((N_HOP, NT)),
            pltpu.SemaphoreType.REGULAR((N_HOP, NT)),
            pltpu.SemaphoreType.DMA((2, NT)),
        ],
        compiler_params=pltpu.CompilerParams(
            collective_id=0, vmem_limit_bytes=64 * 1024 * 1024),
    )(x, w_mat)


# device time: 314509 ns/iter; 1.3229x vs baseline; 1.0652x over previous
import jax
import jax.numpy as jnp
from jax import lax
from jax.experimental import pallas as pl
from jax.experimental.pallas import tpu as pltpu

N_DEV = 4
M_PER = 1024
N_COLS = 8192
HALF = N_COLS // 2
TILE = 1024
NT = HALF // TILE
N_HOP = N_DEV - 1

W_COLS = [ring * HALF + j * TILE
          for _ in range(4) for j in range(NT) for ring in (0, 1)]
N_USES = len(W_COLS)

_GC = 0.7978845608028654


def _gelu(y):
    return 0.5 * y * (1.0 + jnp.tanh(_GC * (y + 0.044715 * y * y * y)))


def _ts(j):
    return slice(j * TILE, (j + 1) * TILE)


def kernel(x, w_mat):
    x = x.astype(jnp.bfloat16)

    def body(x_ref, w_ref, out_ref, cw_ref, ccw_ref,
             cw_send, cw_recv, ccw_send, ccw_recv,
             cw_credit, ccw_credit, w_stage, w_sems, out_sem):
        p = lax.axis_index("i")
        right = lax.rem(p + 1, N_DEV)
        left = lax.rem(p + N_DEV - 1, N_DEV)

        def w_dma(k):
            col = W_COLS[k]
            return pltpu.make_async_copy(
                w_ref.at[:, col:col + TILE], w_stage.at[k % 2],
                w_sems.at[k % 2])

        w_dma(0).start()

        barrier = pltpu.get_barrier_semaphore()
        for nbr in (left, right):
            pl.semaphore_signal(barrier, inc=1, device_id=(nbr,),
                                device_id_type=pl.DeviceIdType.MESH)
        pl.semaphore_wait(barrier, 2)

        def partial_tile(chunk, k):
            if k + 1 < N_USES:
                w_dma(k + 1).start()
            w_dma(k).wait()
            xc = x_ref[pl.ds(chunk * M_PER, M_PER), :]
            w_t = w_stage[k % 2].astype(jnp.bfloat16)
            return jnp.dot(xc, w_t, preferred_element_type=jnp.float32)

        def rdma(ring, h, j):
            ref, ssem, rsem, dst = (
                (cw_ref, cw_send, cw_recv, right) if ring == 0
                else (ccw_ref, ccw_send, ccw_recv, left))
            return pltpu.make_async_remote_copy(
                src_ref=ref.at[h % 2, :, _ts(j)],
                dst_ref=ref.at[(h + 1) % 2, :, _ts(j)],
                send_sem=ssem.at[h, j], recv_sem=rsem.at[h, j],
                device_id=(dst,), device_id_type=pl.DeviceIdType.MESH)

        def give_credit(ring, h, j):
            sem, upstream = ((cw_credit, left) if ring == 0
                             else (ccw_credit, right))
            pl.semaphore_signal(sem.at[h, j], inc=1, device_id=(upstream,),
                                device_id_type=pl.DeviceIdType.MESH)

        def take_credit(ring, h, j):
            sem = cw_credit if ring == 0 else ccw_credit
            pl.semaphore_wait(sem.at[h, j], 1)

        rings = ((0, cw_ref), (1, ccw_ref))

        def use_k(phase, j, ring):
            return phase * 2 * NT + j * 2 + ring

        c0 = {0: lax.rem(p + N_DEV - 1, N_DEV), 1: lax.rem(p + 1, N_DEV)}
        for j in range(NT):
            for ring, ref in rings:
                ref[0, :, _ts(j)] = (
                    partial_tile(c0[ring], use_k(0, j, ring))
                    .astype(jnp.bfloat16))
                rdma(ring, 0, j).start()

        c1 = lax.rem(p + 2, N_DEV)
        for j in range(NT):
            for ring, ref in rings:
                d = rdma(ring, 0, j)
                d.wait_recv()
                ref[1, :, _ts(j)] = (
                    ref[1, :, _ts(j)].astype(jnp.float32)
                    + partial_tile(c1, use_k(1, j, ring))
                ).astype(jnp.bfloat16)
                d.wait_send()
                give_credit(ring, 1, j)
                take_credit(ring, 1, j)
                rdma(ring, 1, j).start()

        c2 = {0: lax.rem(p + 1, N_DEV), 1: lax.rem(p + 3, N_DEV)}
        for j in range(NT):
            for ring, ref in rings:
                d = rdma(ring, 1, j)
                d.wait_recv()
                d.wait_send()
                give_credit(ring, 2, j)
                ref[0, :, _ts(j)] = (
                    ref[0, :, _ts(j)].astype(jnp.float32)
                    + partial_tile(c2[ring], use_k(2, j, ring))
                ).astype(jnp.bfloat16)
                take_credit(ring, 2, j)
                rdma(ring, 2, j).start()

        for j in range(NT):
            for ring, ref in rings:
                d = rdma(ring, 2, j)
                d.wait_recv()
                d.wait_send()
                y = (ref[1, :, _ts(j)].astype(jnp.float32)
                     + partial_tile(p, use_k(3, j, ring)))
                ref[0, :, _ts(j)] = _gelu(y).astype(jnp.bfloat16)
                col = ring * HALF + j * TILE
                pltpu.make_async_copy(
                    ref.at[0, :, _ts(j)],
                    out_ref.at[:, col:col + TILE],
                    out_sem.at[ring, j]).start()

        for j in range(NT):
            for ring, ref in rings:
                col = ring * HALF + j * TILE
                pltpu.make_async_copy(
                    ref.at[0, :, _ts(j)],
                    out_ref.at[:, col:col + TILE],
                    out_sem.at[ring, j]).wait()

    out_shape = jax.ShapeDtypeStruct((M_PER, N_COLS), jnp.bfloat16)
    return pl.pallas_call(
        body,
        out_shape=out_shape,
        in_specs=[pl.BlockSpec(memory_space=pltpu.VMEM),
                  pl.BlockSpec(memory_space=pl.ANY)],
        out_specs=pl.BlockSpec(memory_space=pl.ANY),
        scratch_shapes=[
            pltpu.VMEM((2, M_PER, HALF), jnp.bfloat16),
            pltpu.VMEM((2, M_PER, HALF), jnp.bfloat16),
            pltpu.SemaphoreType.DMA((N_HOP, NT)),
            pltpu.SemaphoreType.DMA((N_HOP, NT)),
            pltpu.SemaphoreType.DMA((N_HOP, NT)),
            pltpu.SemaphoreType.DMA((N_HOP, NT)),
            pltpu.SemaphoreType.REGULAR((N_HOP, NT)),
            pltpu.SemaphoreType.REGULAR((N_HOP, NT)),
            pltpu.VMEM((2, M_PER, TILE), jnp.float32),
            pltpu.SemaphoreType.DMA((2,)),
            pltpu.SemaphoreType.DMA((2, NT)),
        ],
        compiler_params=pltpu.CompilerParams(
            collective_id=0, vmem_limit_bytes=64 * 1024 * 1024),
    )(x, w_mat)


# device time: 300768 ns/iter; 1.3833x vs baseline; 1.0457x over previous
import jax
import jax.numpy as jnp
from jax import lax
from jax.experimental import pallas as pl
from jax.experimental.pallas import tpu as pltpu

N_DEV = 4
M_PER = 1024
K_PER = 1024
N_COLS = 8192
HALF = N_COLS // 2
TILE = 1024
NT = HALF // TILE
N_HOP = N_DEV - 1

W_COLS = [ring * HALF + j * TILE
          for _ in range(4) for j in range(NT) for ring in (0, 1)]
N_USES = len(W_COLS)

_GC = 0.7978845608028654


def _gelu(y):
    return 0.5 * y * (1.0 + jnp.tanh(_GC * (y + 0.044715 * y * y * y)))


def _ts(j):
    return slice(j * TILE, (j + 1) * TILE)


def kernel(x, w_mat):

    def body(x_ref, w_ref, out_ref, cw_ref, ccw_ref,
             cw_send, cw_recv, ccw_send, ccw_recv,
             cw_credit, ccw_credit,
             xb_ref, x_stage, x_sem, w_stage, w_sems, out_sem):
        p = lax.axis_index("i")
        right = lax.rem(p + 1, N_DEV)
        left = lax.rem(p + N_DEV - 1, N_DEV)

        def w_dma(k):
            col = W_COLS[k]
            return pltpu.make_async_copy(
                w_ref.at[:, col:col + TILE], w_stage.at[k % 2],
                w_sems.at[k % 2])

        def x_dma(chunk):
            return pltpu.make_async_copy(
                x_ref.at[pl.ds(chunk * M_PER, M_PER), :], x_stage,
                x_sem)

        x_chunks = [lax.rem(p + c, N_DEV) for c in (3, 1, 2, 0)]

        def x_convert(i):
            c = x_chunks[i]
            x_dma(c).wait()
            xb_ref[pl.ds(c * M_PER, M_PER), :] = (
                x_stage[...].astype(jnp.bfloat16))
            if i + 1 < len(x_chunks):
                x_dma(x_chunks[i + 1]).start()

        w_dma(0).start()
        x_dma(x_chunks[0]).start()

        barrier = pltpu.get_barrier_semaphore()
        for nbr in (left, right):
            pl.semaphore_signal(barrier, inc=1, device_id=(nbr,),
                                device_id_type=pl.DeviceIdType.MESH)
        pl.semaphore_wait(barrier, 2)

        x_convert(0)

        def partial_tile(chunk, k):
            if k + 1 < N_USES:
                w_dma(k + 1).start()
            w_dma(k).wait()
            xc = xb_ref[pl.ds(chunk * M_PER, M_PER), :]
            w_t = w_stage[k % 2].astype(jnp.bfloat16)
            return jnp.dot(xc, w_t, preferred_element_type=jnp.float32)

        def rdma(ring, h, j):
            ref, ssem, rsem, dst = (
                (cw_ref, cw_send, cw_recv, right) if ring == 0
                else (ccw_ref, ccw_send, ccw_recv, left))
            return pltpu.make_async_remote_copy(
                src_ref=ref.at[h % 2, :, _ts(j)],
                dst_ref=ref.at[(h + 1) % 2, :, _ts(j)],
                send_sem=ssem.at[h, j], recv_sem=rsem.at[h, j],
                device_id=(dst,), device_id_type=pl.DeviceIdType.MESH)

        def give_credit(ring, h, j):
            sem, upstream = ((cw_credit, left) if ring == 0
                             else (ccw_credit, right))
            pl.semaphore_signal(sem.at[h, j], inc=1, device_id=(upstream,),
                                device_id_type=pl.DeviceIdType.MESH)

        def take_credit(ring, h, j):
            sem = cw_credit if ring == 0 else ccw_credit
            pl.semaphore_wait(sem.at[h, j], 1)

        rings = ((0, cw_ref), (1, ccw_ref))

        def use_k(phase, j, ring):
            return phase * 2 * NT + j * 2 + ring

        c0 = {0: x_chunks[0], 1: x_chunks[1]}
        for j in range(NT):
            for ring, ref in rings:
                if j == 0 and ring == 1:
                    x_convert(1)
                ref[0, :, _ts(j)] = (
                    partial_tile(c0[ring], use_k(0, j, ring))
                    .astype(jnp.bfloat16))
                rdma(ring, 0, j).start()

        x_convert(2)
        x_convert(3)

        c1 = x_chunks[2]
        for j in range(NT):
            for ring, ref in rings:
                pt = partial_tile(c1, use_k(1, j, ring))
                d = rdma(ring, 0, j)
                d.wait_recv()
                ref[1, :, _ts(j)] = (
                    ref[1, :, _ts(j)].astype(jnp.float32) + pt
                ).astype(jnp.bfloat16)
                d.wait_send()
                give_credit(ring, 1, j)
                take_credit(ring, 1, j)
                rdma(ring, 1, j).start()

        c2 = {0: x_chunks[1], 1: x_chunks[0]}
        for j in range(NT):
            for ring, ref in rings:
                pt = partial_tile(c2[ring], use_k(2, j, ring))
                d = rdma(ring, 1, j)
                d.wait_recv()
                d.wait_send()
                give_credit(ring, 2, j)
                ref[0, :, _ts(j)] = (
                    ref[0, :, _ts(j)].astype(jnp.float32) + pt
                ).astype(jnp.bfloat16)
                take_credit(ring, 2, j)
                rdma(ring, 2, j).start()

        c3 = x_chunks[3]
        for j in range(NT):
            for ring, ref in rings:
                pt = partial_tile(c3, use_k(3, j, ring))
                d = rdma(ring, 2, j)
                d.wait_recv()
                d.wait_send()
                y = ref[1, :, _ts(j)].astype(jnp.float32) + pt
                ref[0, :, _ts(j)] = _gelu(y).astype(jnp.bfloat16)
                col = ring * HALF + j * TILE
                pltpu.make_async_copy(
                    ref.at[0, :, _ts(j)],
                    out_ref.at[:, col:col + TILE],
                    out_sem.at[ring, j]).start()

        for j in range(NT):
            for ring, ref in rings:
                col = ring * HALF + j * TILE
                pltpu.make_async_copy(
                    ref.at[0, :, _ts(j)],
                    out_ref.at[:, col:col + TILE],
                    out_sem.at[ring, j]).wait()

    out_shape = jax.ShapeDtypeStruct((M_PER, N_COLS), jnp.bfloat16)
    return pl.pallas_call(
        body,
        out_shape=out_shape,
        in_specs=[pl.BlockSpec(memory_space=pl.ANY),
                  pl.BlockSpec(memory_space=pl.ANY)],
        out_specs=pl.BlockSpec(memory_space=pl.ANY),
        scratch_shapes=[
            pltpu.VMEM((2, M_PER, HALF), jnp.bfloat16),
            pltpu.VMEM((2, M_PER, HALF), jnp.bfloat16),
            pltpu.SemaphoreType.DMA((N_HOP, NT)),
            pltpu.SemaphoreType.DMA((N_HOP, NT)),
            pltpu.SemaphoreType.DMA((N_HOP, NT)),
            pltpu.SemaphoreType.DMA((N_HOP, NT)),
            pltpu.SemaphoreType.REGULAR((N_HOP, NT)),
            pltpu.SemaphoreType.REGULAR((N_HOP, NT)),
            pltpu.VMEM((4 * M_PER, K_PER), jnp.bfloat16),
            pltpu.VMEM((M_PER, K_PER), jnp.float32),
            pltpu.SemaphoreType.DMA,
            pltpu.VMEM((2, K_PER, TILE), jnp.float32),
            pltpu.SemaphoreType.DMA((2,)),
            pltpu.SemaphoreType.DMA((2, NT)),
        ],
        compiler_params=pltpu.CompilerParams(
            collective_id=0, vmem_limit_bytes=64 * 1024 * 1024),
    )(x, w_mat)
